# baseline (speedup 1.0000x reference)
import jax
import jax.numpy as jnp
from jax.experimental import pallas as pl

K_NN = 20


def _knn_feat_kernel(xt_blk_ref, x_all_ref, w1_ref, wd_ref, bias_ref, out_ref):
    xt_blk = xt_blk_ref[...]
    x_all = x_all_ref[...]
    R = xt_blk.shape[0]
    N = x_all.shape[1]

    r2 = jnp.sum(xt_blk * xt_blk, axis=1, keepdims=True)
    c2 = jnp.sum(x_all * x_all, axis=0, keepdims=True)
    d = 2.0 * jnp.dot(xt_blk, x_all, preferred_element_type=jnp.float32)
    d = d - r2 - c2

    iota = jax.lax.broadcasted_iota(jnp.int32, (R, N), 1)
    neg_inf = jnp.float32(-jnp.inf)

    def body(_, dcur):
        a = jnp.argmax(dcur, axis=1)
        return jnp.where(iota == a[:, None], neg_inf, dcur)

    d_final = jax.lax.fori_loop(0, K_NN, body, d)
    mask = (d_final == neg_inf).astype(jnp.float32)

    g = jax.lax.dot_general(mask, x_all, (((1,), (1,)), ((), ())),
                            preferred_element_type=jnp.float32)
    g = g * jnp.float32(1.0 / K_NN)

    o = jax.lax.dot_general(g, w1_ref[...], (((1,), (1,)), ((), ())),
                            preferred_element_type=jnp.float32)
    o = o + jax.lax.dot_general(xt_blk, wd_ref[...], (((1,), (1,)), ((), ())),
                                preferred_element_type=jnp.float32)
    out_ref[...] = o + bias_ref[...]


def kernel(x, W, b):
    B, C, N = x.shape
    O = W.shape[0]
    Wm = W[:, :, 0, 0]
    w1 = Wm[:, :C]
    wd = Wm[:, C:] - w1
    xt = jnp.transpose(x, (0, 2, 1))
    bias = b[None, :]

    R = min(256, N)
    grid = (B, N // R)

    out = pl.pallas_call(
        _knn_feat_kernel,
        grid=grid,
        in_specs=[
            pl.BlockSpec((None, R, C), lambda bb, i: (bb, i, 0)),
            pl.BlockSpec((None, C, N), lambda bb, i: (bb, 0, 0)),
            pl.BlockSpec((O, C), lambda bb, i: (0, 0)),
            pl.BlockSpec((O, C), lambda bb, i: (0, 0)),
            pl.BlockSpec((1, O), lambda bb, i: (0, 0)),
        ],
        out_specs=pl.BlockSpec((None, R, O), lambda bb, i: (bb, i, 0)),
        out_shape=jax.ShapeDtypeStruct((B, N, O), jnp.float32),
    )(xt, x, w1, wd, bias)

    return jnp.transpose(out, (0, 2, 1))


# speedup vs baseline: 1.0065x; 1.0065x over previous
import jax
import jax.numpy as jnp
from jax.experimental import pallas as pl

K_NN = 20


def _knn_feat_kernel(xt_blk_ref, x_all_ref, w1_ref, wd_ref, bias_ref, out_ref):
    xt_blk = xt_blk_ref[...]
    x_all = x_all_ref[...]
    R = xt_blk.shape[0]
    N = x_all.shape[1]

    r2 = jnp.sum(xt_blk * xt_blk, axis=1, keepdims=True)
    c2 = jnp.sum(x_all * x_all, axis=0, keepdims=True)
    d = 2.0 * jnp.dot(xt_blk, x_all, preferred_element_type=jnp.float32)
    d = d - r2 - c2

    iota = jax.lax.broadcasted_iota(jnp.int32, (R, N), 1)
    neg_inf = jnp.float32(-jnp.inf)

    def body(_, carry):
        dcur, m = carry
        cand = jnp.where(dcur == m, iota, jnp.int32(N))
        a = jnp.min(cand, axis=1, keepdims=True)
        dnew = jnp.where(iota == a, neg_inf, dcur)
        return dnew, jnp.max(dnew, axis=1, keepdims=True)

    m0 = jnp.max(d, axis=1, keepdims=True)
    d_final, _ = jax.lax.fori_loop(0, K_NN, body, (d, m0))
    mask = (d_final == neg_inf).astype(jnp.float32)

    g = jax.lax.dot_general(mask, x_all, (((1,), (1,)), ((), ())),
                            preferred_element_type=jnp.float32)
    g = g * jnp.float32(1.0 / K_NN)

    o = jax.lax.dot_general(g, w1_ref[...], (((1,), (1,)), ((), ())),
                            preferred_element_type=jnp.float32)
    o = o + jax.lax.dot_general(xt_blk, wd_ref[...], (((1,), (1,)), ((), ())),
                                preferred_element_type=jnp.float32)
    out_ref[...] = o + bias_ref[...]


def kernel(x, W, b):
    B, C, N = x.shape
    O = W.shape[0]
    Wm = W[:, :, 0, 0]
    w1 = Wm[:, :C]
    wd = Wm[:, C:] - w1
    xt = jnp.transpose(x, (0, 2, 1))
    bias = b[None, :]

    R = min(256, N)
    grid = (B, N // R)

    out = pl.pallas_call(
        _knn_feat_kernel,
        grid=grid,
        in_specs=[
            pl.BlockSpec((None, R, C), lambda bb, i: (bb, i, 0)),
            pl.BlockSpec((None, C, N), lambda bb, i: (bb, 0, 0)),
            pl.BlockSpec((O, C), lambda bb, i: (0, 0)),
            pl.BlockSpec((O, C), lambda bb, i: (0, 0)),
            pl.BlockSpec((1, O), lambda bb, i: (0, 0)),
        ],
        out_specs=pl.BlockSpec((None, R, O), lambda bb, i: (bb, i, 0)),
        out_shape=jax.ShapeDtypeStruct((B, N, O), jnp.float32),
    )(xt, x, w1, wd, bias)

    return jnp.transpose(out, (0, 2, 1))


# speedup vs baseline: 1.7266x; 1.7154x over previous
import jax
import jax.numpy as jnp
from jax.experimental import pallas as pl

K_NN = 20


def _knn_feat_kernel(xt_blk_ref, x_all_ref, w1_ref, wd_ref, bias_ref, out_ref):
    xt_blk = xt_blk_ref[...]
    x_all = x_all_ref[...]
    R = xt_blk.shape[0]
    N = x_all.shape[1]

    r2 = jnp.sum(xt_blk * xt_blk, axis=1, keepdims=True)
    c2 = jnp.sum(x_all * x_all, axis=0, keepdims=True)
    d = 2.0 * jnp.dot(xt_blk, x_all, preferred_element_type=jnp.float32)
    d = d - r2 - c2

    iota = jax.lax.broadcasted_iota(jnp.int32, (R, N), 1)
    neg_inf = jnp.float32(-jnp.inf)

    def body(_, dcur):
        m = jnp.max(dcur, axis=1, keepdims=True)
        return jnp.where(dcur == m, neg_inf, dcur)

    d_final = jax.lax.fori_loop(0, K_NN, body, d)
    mask = (d_final == neg_inf).astype(jnp.float32)

    g = jax.lax.dot_general(mask, x_all, (((1,), (1,)), ((), ())),
                            preferred_element_type=jnp.float32)
    g = g * jnp.float32(1.0 / K_NN)

    o = jax.lax.dot_general(g, w1_ref[...], (((1,), (1,)), ((), ())),
                            preferred_element_type=jnp.float32)
    o = o + jax.lax.dot_general(xt_blk, wd_ref[...], (((1,), (1,)), ((), ())),
                                preferred_element_type=jnp.float32)
    out_ref[...] = o + bias_ref[...]


def kernel(x, W, b):
    B, C, N = x.shape
    O = W.shape[0]
    Wm = W[:, :, 0, 0]
    w1 = Wm[:, :C]
    wd = Wm[:, C:] - w1
    xt = jnp.transpose(x, (0, 2, 1))
    bias = b[None, :]

    R = min(256, N)
    grid = (B, N // R)

    out = pl.pallas_call(
        _knn_feat_kernel,
        grid=grid,
        in_specs=[
            pl.BlockSpec((None, R, C), lambda bb, i: (bb, i, 0)),
            pl.BlockSpec((None, C, N), lambda bb, i: (bb, 0, 0)),
            pl.BlockSpec((O, C), lambda bb, i: (0, 0)),
            pl.BlockSpec((O, C), lambda bb, i: (0, 0)),
            pl.BlockSpec((1, O), lambda bb, i: (0, 0)),
        ],
        out_specs=pl.BlockSpec((None, R, O), lambda bb, i: (bb, i, 0)),
        out_shape=jax.ShapeDtypeStruct((B, N, O), jnp.float32),
    )(xt, x, w1, wd, bias)

    return jnp.transpose(out, (0, 2, 1))


# speedup vs baseline: 1.8248x; 1.0569x over previous
import jax
import jax.numpy as jnp
from jax.experimental import pallas as pl

K_NN = 20


def _knn_feat_kernel(xt_blk_ref, x_all_ref, w1_ref, wd_ref, bias_ref, out_ref):
    xt_blk = xt_blk_ref[...]
    x_all = x_all_ref[...]
    R = xt_blk.shape[0]
    N = x_all.shape[1]

    r2 = jnp.sum(xt_blk * xt_blk, axis=1, keepdims=True)
    c2 = jnp.sum(x_all * x_all, axis=0, keepdims=True)
    d = 2.0 * jnp.dot(xt_blk, x_all, preferred_element_type=jnp.float32)
    d = d - r2 - c2

    iota = jax.lax.broadcasted_iota(jnp.int32, (R, N), 1)
    neg_inf = jnp.float32(-jnp.inf)

    def body(_, carry):
        dcur, m = carry
        dnew = jnp.where(dcur == m, neg_inf, dcur)
        return dnew, jnp.max(dnew, axis=1, keepdims=True)

    m0 = jnp.max(d, axis=1, keepdims=True)
    d_final, _ = jax.lax.fori_loop(0, K_NN, body, (d, m0))
    mask = (d_final == neg_inf).astype(jnp.float32)

    g = jax.lax.dot_general(mask, x_all, (((1,), (1,)), ((), ())),
                            preferred_element_type=jnp.float32)
    g = g * jnp.float32(1.0 / K_NN)

    o = jax.lax.dot_general(g, w1_ref[...], (((1,), (1,)), ((), ())),
                            preferred_element_type=jnp.float32)
    o = o + jax.lax.dot_general(xt_blk, wd_ref[...], (((1,), (1,)), ((), ())),
                                preferred_element_type=jnp.float32)
    out_ref[...] = o + bias_ref[...]


def kernel(x, W, b):
    B, C, N = x.shape
    O = W.shape[0]
    Wm = W[:, :, 0, 0]
    w1 = Wm[:, :C]
    wd = Wm[:, C:] - w1
    xt = jnp.transpose(x, (0, 2, 1))
    bias = b[None, :]

    R = min(256, N)
    grid = (B, N // R)

    out = pl.pallas_call(
        _knn_feat_kernel,
        grid=grid,
        in_specs=[
            pl.BlockSpec((None, R, C), lambda bb, i: (bb, i, 0)),
            pl.BlockSpec((None, C, N), lambda bb, i: (bb, 0, 0)),
            pl.BlockSpec((O, C), lambda bb, i: (0, 0)),
            pl.BlockSpec((O, C), lambda bb, i: (0, 0)),
            pl.BlockSpec((1, O), lambda bb, i: (0, 0)),
        ],
        out_specs=pl.BlockSpec((None, R, O), lambda bb, i: (bb, i, 0)),
        out_shape=jax.ShapeDtypeStruct((B, N, O), jnp.float32),
    )(xt, x, w1, wd, bias)

    return jnp.transpose(out, (0, 2, 1))


# speedup vs baseline: 3.9324x; 2.1549x over previous
import jax
import jax.numpy as jnp
from jax.experimental import pallas as pl
from jax.experimental.pallas import tpu as pltpu

K_NN = 20


def _knn_feat_kernel(xt_cur_ref, x_all_ref, xt_prev_ref, w1_ref, wd_ref,
                     bias_ref, out_ref, dbuf_ref):
    i = pl.program_id(1)
    nb = pl.num_programs(1) - 1
    neg_inf = jnp.float32(-jnp.inf)

    @pl.when(i < nb)
    def _compute_dist():
        xt_blk = xt_cur_ref[...]
        x_all = x_all_ref[...]
        r2 = jnp.sum(xt_blk * xt_blk, axis=1, keepdims=True)
        c2 = jnp.sum(x_all * x_all, axis=0, keepdims=True)
        d = 2.0 * jnp.dot(xt_blk, x_all, preferred_element_type=jnp.float32)
        dbuf_ref[i % 2] = d - r2 - c2

    @pl.when(i > 0)
    def _select_and_project():
        d = dbuf_ref[(i - 1) % 2]
        m = jnp.max(d, axis=1, keepdims=True)
        for j in range(K_NN):
            d = jnp.where(d == m, neg_inf, d)
            if j < K_NN - 1:
                m = jnp.max(d, axis=1, keepdims=True)
        mask = (d == neg_inf).astype(jnp.float32)

        g = jax.lax.dot_general(mask, x_all_ref[...], (((1,), (1,)), ((), ())),
                                preferred_element_type=jnp.float32)
        g = g * jnp.float32(1.0 / K_NN)

        xt_blk = xt_prev_ref[...]
        o = jax.lax.dot_general(g, w1_ref[...], (((1,), (1,)), ((), ())),
                                preferred_element_type=jnp.float32)
        o = o + jax.lax.dot_general(xt_blk, wd_ref[...],
                                    (((1,), (1,)), ((), ())),
                                    preferred_element_type=jnp.float32)
        out_ref[...] = o + bias_ref[...]


def kernel(x, W, b):
    B, C, N = x.shape
    O = W.shape[0]
    Wm = W[:, :, 0, 0]
    w1 = Wm[:, :C]
    wd = Wm[:, C:] - w1
    xt = jnp.transpose(x, (0, 2, 1))
    bias = b[None, :]

    R = min(256, N)
    nb = N // R
    grid = (B, nb + 1)

    out = pl.pallas_call(
        _knn_feat_kernel,
        grid=grid,
        in_specs=[
            pl.BlockSpec((None, R, C), lambda bb, i: (bb, jnp.minimum(i, nb - 1), 0)),
            pl.BlockSpec((None, C, N), lambda bb, i: (bb, 0, 0)),
            pl.BlockSpec((None, R, C), lambda bb, i: (bb, jnp.maximum(i - 1, 0), 0)),
            pl.BlockSpec((O, C), lambda bb, i: (0, 0)),
            pl.BlockSpec((O, C), lambda bb, i: (0, 0)),
            pl.BlockSpec((1, O), lambda bb, i: (0, 0)),
        ],
        out_specs=pl.BlockSpec((None, R, O), lambda bb, i: (bb, jnp.maximum(i - 1, 0), 0)),
        out_shape=jax.ShapeDtypeStruct((B, N, O), jnp.float32),
        scratch_shapes=[pltpu.VMEM((2, R, N), jnp.float32)],
    )(xt, x, xt, w1, wd, bias)

    return jnp.transpose(out, (0, 2, 1))


# speedup vs baseline: 3.9912x; 1.0150x over previous
import jax
import jax.numpy as jnp
from jax.experimental import pallas as pl
from jax.experimental.pallas import tpu as pltpu

K_NN = 20


def _knn_feat_kernel(xt_cur_ref, x_all_ref, xt_prev_ref, w1_ref, wd_ref,
                     bias_ref, out_ref, dbuf_ref):
    i = pl.program_id(1)
    nb = pl.num_programs(1) - 1
    neg_inf = jnp.float32(-jnp.inf)

    @pl.when(i < nb)
    def _compute_dist():
        xt_blk = xt_cur_ref[...]
        x_all = x_all_ref[...]
        r2 = jnp.sum(xt_blk * xt_blk, axis=1, keepdims=True)
        c2 = jnp.sum(x_all * x_all, axis=0, keepdims=True)
        d = 2.0 * jnp.dot(xt_blk, x_all, preferred_element_type=jnp.float32)
        dbuf_ref[i % 2] = d - r2 - c2

    @pl.when(i > 0)
    def _select_and_project():
        d = dbuf_ref[(i - 1) % 2]
        m = jnp.max(d, axis=1, keepdims=True)
        for j in range(K_NN):
            d = jnp.where(d == m, neg_inf, d)
            if j < K_NN - 1:
                m = jnp.max(d, axis=1, keepdims=True)
        mask = (d == neg_inf).astype(jnp.float32)

        g = jax.lax.dot_general(mask, x_all_ref[...], (((1,), (1,)), ((), ())),
                                preferred_element_type=jnp.float32)
        g = g * jnp.float32(1.0 / K_NN)

        xt_blk = xt_prev_ref[...]
        o = jax.lax.dot_general(g, w1_ref[...], (((1,), (1,)), ((), ())),
                                preferred_element_type=jnp.float32)
        o = o + jax.lax.dot_general(xt_blk, wd_ref[...],
                                    (((1,), (1,)), ((), ())),
                                    preferred_element_type=jnp.float32)
        out_ref[...] = o + bias_ref[...]


def kernel(x, W, b):
    B, C, N = x.shape
    O = W.shape[0]
    Wm = W[:, :, 0, 0]
    w1 = Wm[:, :C]
    wd = Wm[:, C:] - w1
    xt = jnp.transpose(x, (0, 2, 1))
    bias = b[None, :]

    R = min(512, N)
    nb = N // R
    grid = (B, nb + 1)

    out = pl.pallas_call(
        _knn_feat_kernel,
        grid=grid,
        in_specs=[
            pl.BlockSpec((None, R, C), lambda bb, i: (bb, jnp.minimum(i, nb - 1), 0)),
            pl.BlockSpec((None, C, N), lambda bb, i: (bb, 0, 0)),
            pl.BlockSpec((None, R, C), lambda bb, i: (bb, jnp.maximum(i - 1, 0), 0)),
            pl.BlockSpec((O, C), lambda bb, i: (0, 0)),
            pl.BlockSpec((O, C), lambda bb, i: (0, 0)),
            pl.BlockSpec((1, O), lambda bb, i: (0, 0)),
        ],
        out_specs=pl.BlockSpec((None, R, O), lambda bb, i: (bb, jnp.maximum(i - 1, 0), 0)),
        out_shape=jax.ShapeDtypeStruct((B, N, O), jnp.float32),
        scratch_shapes=[pltpu.VMEM((2, R, N), jnp.float32)],
    )(xt, x, xt, w1, wd, bias)

    return jnp.transpose(out, (0, 2, 1))


# speedup vs baseline: 4.4121x; 1.1054x over previous
import jax
import jax.numpy as jnp
from jax.experimental import pallas as pl
from jax.experimental.pallas import tpu as pltpu

K_NN = 20


def _knn_feat_kernel(x_cur_ref, x_all_ref, x_prev_ref, w1_ref, wd_ref,
                     bias_ref, out_ref, dbuf_ref):
    i = pl.program_id(1)
    nb = pl.num_programs(1) - 1
    neg_inf = jnp.float32(-jnp.inf)

    @pl.when(i < nb)
    def _compute_dist():
        x_blk = x_cur_ref[...]
        x_all = x_all_ref[...]
        c2 = jnp.sum(x_all * x_all, axis=0, keepdims=True)
        d = 2.0 * jax.lax.dot_general(
            x_blk, x_all, (((0,), (0,)), ((), ())),
            preferred_element_type=jnp.float32)
        dbuf_ref[i % 2] = d - c2

    @pl.when(i > 0)
    def _select_and_project():
        d = dbuf_ref[(i - 1) % 2]
        m = jnp.max(d, axis=1, keepdims=True)
        for j in range(K_NN):
            d = jnp.where(d == m, neg_inf, d)
            if j < K_NN - 1:
                m = jnp.max(d, axis=1, keepdims=True)
        mask = (d == neg_inf).astype(jnp.float32)

        g = jax.lax.dot_general(x_all_ref[...], mask,
                                (((1,), (1,)), ((), ())),
                                preferred_element_type=jnp.float32)
        g = g * jnp.float32(1.0 / K_NN)

        o = jax.lax.dot_general(w1_ref[...], g, (((1,), (0,)), ((), ())),
                                preferred_element_type=jnp.float32)
        o = o + jax.lax.dot_general(wd_ref[...], x_prev_ref[...],
                                    (((1,), (0,)), ((), ())),
                                    preferred_element_type=jnp.float32)
        out_ref[...] = o + bias_ref[...]


def kernel(x, W, b):
    B, C, N = x.shape
    O = W.shape[0]
    Wm = W[:, :, 0, 0]
    w1 = Wm[:, :C]
    wd = Wm[:, C:] - w1
    bias = b[:, None]

    R = min(512, N)
    nb = N // R
    grid = (B, nb + 1)

    return pl.pallas_call(
        _knn_feat_kernel,
        grid=grid,
        in_specs=[
            pl.BlockSpec((None, C, R), lambda bb, i: (bb, 0, jnp.minimum(i, nb - 1))),
            pl.BlockSpec((None, C, N), lambda bb, i: (bb, 0, 0)),
            pl.BlockSpec((None, C, R), lambda bb, i: (bb, 0, jnp.maximum(i - 1, 0))),
            pl.BlockSpec((O, C), lambda bb, i: (0, 0)),
            pl.BlockSpec((O, C), lambda bb, i: (0, 0)),
            pl.BlockSpec((O, 1), lambda bb, i: (0, 0)),
        ],
        out_specs=pl.BlockSpec((None, O, R), lambda bb, i: (bb, 0, jnp.maximum(i - 1, 0))),
        out_shape=jax.ShapeDtypeStruct((B, O, N), jnp.float32),
        scratch_shapes=[pltpu.VMEM((2, R, N), jnp.float32)],
    )(x, x, x, w1, wd, bias)


# speedup vs baseline: 4.4301x; 1.0041x over previous
import jax
import jax.numpy as jnp
from jax.experimental import pallas as pl
from jax.experimental.pallas import tpu as pltpu

K_NN = 20


def _knn_feat_kernel(x_cur_ref, x_all_cur_ref, x_prev_ref, x_all_prev_ref,
                     w1_ref, wd_ref, bias_ref, out_ref, dbuf_ref):
    i = pl.program_id(0)
    nsteps = pl.num_programs(0) - 1
    neg_inf = jnp.float32(-jnp.inf)

    @pl.when(i < nsteps)
    def _compute_dist():
        x_blk = x_cur_ref[...]
        x_all = x_all_cur_ref[...]
        c2 = jnp.sum(x_all * x_all, axis=0, keepdims=True)
        d = 2.0 * jax.lax.dot_general(
            x_blk, x_all, (((0,), (0,)), ((), ())),
            preferred_element_type=jnp.float32)
        dbuf_ref[i % 2] = d - c2

    @pl.when(i > 0)
    def _select_and_project():
        d = dbuf_ref[(i - 1) % 2]
        m = jnp.max(d, axis=1, keepdims=True)
        for j in range(K_NN):
            d = jnp.where(d == m, neg_inf, d)
            if j < K_NN - 1:
                m = jnp.max(d, axis=1, keepdims=True)
        mask = (d == neg_inf).astype(jnp.float32)

        g = jax.lax.dot_general(x_all_prev_ref[...], mask,
                                (((1,), (1,)), ((), ())),
                                preferred_element_type=jnp.float32)
        g = g * jnp.float32(1.0 / K_NN)

        o = jax.lax.dot_general(w1_ref[...], g, (((1,), (0,)), ((), ())),
                                preferred_element_type=jnp.float32)
        o = o + jax.lax.dot_general(wd_ref[...], x_prev_ref[...],
                                    (((1,), (0,)), ((), ())),
                                    preferred_element_type=jnp.float32)
        out_ref[...] = o + bias_ref[...]


def kernel(x, W, b):
    B, C, N = x.shape
    O = W.shape[0]
    Wm = W[:, :, 0, 0]
    w1 = Wm[:, :C]
    wd = Wm[:, C:] - w1
    bias = b[:, None]

    R = min(512, N)
    nb = N // R
    nblocks = B * nb
    grid = (nblocks + 1,)

    def cur_blk(t):
        f = jnp.minimum(t, nblocks - 1)
        return f // nb, 0, f % nb

    def prev_blk(t):
        g = jnp.maximum(t - 1, 0)
        return g // nb, 0, g % nb

    return pl.pallas_call(
        _knn_feat_kernel,
        grid=grid,
        in_specs=[
            pl.BlockSpec((None, C, R), cur_blk),
            pl.BlockSpec((None, C, N), lambda t: (jnp.minimum(t, nblocks - 1) // nb, 0, 0)),
            pl.BlockSpec((None, C, R), prev_blk),
            pl.BlockSpec((None, C, N), lambda t: (jnp.maximum(t - 1, 0) // nb, 0, 0)),
            pl.BlockSpec((O, C), lambda t: (0, 0)),
            pl.BlockSpec((O, C), lambda t: (0, 0)),
            pl.BlockSpec((O, 1), lambda t: (0, 0)),
        ],
        out_specs=pl.BlockSpec((None, O, R), prev_blk),
        out_shape=jax.ShapeDtypeStruct((B, O, N), jnp.float32),
        scratch_shapes=[pltpu.VMEM((2, R, N), jnp.float32)],
    )(x, x, x, x, w1, wd, bias)


# speedup vs baseline: 4.6305x; 1.0452x over previous
import jax
import jax.numpy as jnp
from jax.experimental import pallas as pl
from jax.experimental.pallas import tpu as pltpu

K_NN = 20


def _knn_feat_kernel(x_cur_ref, x_all_cur_ref, x_prev_ref, x_all_prev_ref,
                     w1_ref, wd_ref, bias_ref, out_ref, dbuf_ref):
    i = pl.program_id(0)
    nsteps = pl.num_programs(0) - 1
    neg_inf = jnp.float32(-jnp.inf)

    @pl.when(i < nsteps)
    def _compute_dist():
        x_blk = x_cur_ref[...]
        x_all = x_all_cur_ref[...]
        c2 = jnp.sum(x_all * x_all, axis=0, keepdims=True)
        d = 2.0 * jax.lax.dot_general(
            x_blk, x_all, (((0,), (0,)), ((), ())),
            preferred_element_type=jnp.float32)
        dbuf_ref[i % 2] = d - c2

    @pl.when(i > 0)
    def _select_and_project():
        d = dbuf_ref[(i - 1) % 2]
        m = jnp.max(d, axis=1, keepdims=True)
        for _ in range(K_NN - 1):
            m = jnp.max(jnp.where(d < m, d, neg_inf), axis=1, keepdims=True)
        mask = (d >= m).astype(jnp.float32)

        g = jax.lax.dot_general(x_all_prev_ref[...], mask,
                                (((1,), (1,)), ((), ())),
                                preferred_element_type=jnp.float32)
        g = g * jnp.float32(1.0 / K_NN)

        o = jax.lax.dot_general(w1_ref[...], g, (((1,), (0,)), ((), ())),
                                preferred_element_type=jnp.float32)
        o = o + jax.lax.dot_general(wd_ref[...], x_prev_ref[...],
                                    (((1,), (0,)), ((), ())),
                                    preferred_element_type=jnp.float32)
        out_ref[...] = o + bias_ref[...]


def kernel(x, W, b):
    B, C, N = x.shape
    O = W.shape[0]
    Wm = W[:, :, 0, 0]
    w1 = Wm[:, :C]
    wd = Wm[:, C:] - w1
    bias = b[:, None]

    R = min(512, N)
    nb = N // R
    nblocks = B * nb
    grid = (nblocks + 1,)

    def cur_blk(t):
        f = jnp.minimum(t, nblocks - 1)
        return f // nb, 0, f % nb

    def prev_blk(t):
        g = jnp.maximum(t - 1, 0)
        return g // nb, 0, g % nb

    return pl.pallas_call(
        _knn_feat_kernel,
        grid=grid,
        in_specs=[
            pl.BlockSpec((None, C, R), cur_blk),
            pl.BlockSpec((None, C, N), lambda t: (jnp.minimum(t, nblocks - 1) // nb, 0, 0)),
            pl.BlockSpec((None, C, R), prev_blk),
            pl.BlockSpec((None, C, N), lambda t: (jnp.maximum(t - 1, 0) // nb, 0, 0)),
            pl.BlockSpec((O, C), lambda t: (0, 0)),
            pl.BlockSpec((O, C), lambda t: (0, 0)),
            pl.BlockSpec((O, 1), lambda t: (0, 0)),
        ],
        out_specs=pl.BlockSpec((None, O, R), prev_blk),
        out_shape=jax.ShapeDtypeStruct((B, O, N), jnp.float32),
        scratch_shapes=[pltpu.VMEM((2, R, N), jnp.float32)],
    )(x, x, x, x, w1, wd, bias)


# speedup vs baseline: 7.8705x; 1.6997x over previous
import jax
import jax.numpy as jnp
from jax.experimental import pallas as pl
from jax.experimental.pallas import tpu as pltpu

K_NN = 20


N_SLOTS = 5
LANES = 128


def _knn_feat_kernel(x_cur_ref, x_all_cur_ref, x_prev_ref, x_all_prev_ref,
                     w1_ref, wd_ref, bias_ref, out_ref, dbuf_ref, mref):
    i = pl.program_id(0)
    nsteps = pl.num_programs(0) - 1
    neg_inf = jnp.float32(-jnp.inf)

    @pl.when(i < nsteps)
    def _compute_dist():
        x_blk = x_cur_ref[...]
        x_all = x_all_cur_ref[...]
        c2 = jnp.sum(x_all * x_all, axis=0, keepdims=True)
        d = 2.0 * jax.lax.dot_general(
            x_blk, x_all, (((0,), (0,)), ((), ())),
            preferred_element_type=jnp.float32)
        dbuf_ref[i % 2] = d - c2

    @pl.when(i > 0)
    def _select_and_project():
        d = dbuf_ref[(i - 1) % 2]
        N = d.shape[1]
        s = [None] * N_SLOTS
        for t in range(N // LANES):
            v = d[:, t * LANES:(t + 1) * LANES]
            for q in range(N_SLOTS):
                if s[q] is None:
                    s[q] = v
                    break
                hi = jnp.maximum(s[q], v)
                v = jnp.minimum(s[q], v)
                s[q] = hi
        for q in range(N_SLOTS):
            if s[q] is None:
                s[q] = jnp.full_like(s[0], neg_inf)

        m = jnp.max(s[0], axis=1, keepdims=True)
        for j in range(K_NN - 1):
            c = s[0] == m
            for q in range(N_SLOTS - 1):
                s[q] = jnp.where(c, s[q + 1], s[q])
            s[N_SLOTS - 1] = jnp.where(c, neg_inf, s[N_SLOTS - 1])
            m = jnp.max(s[0], axis=1, keepdims=True)

        cnt = jnp.sum((d >= m).astype(jnp.float32), axis=1)
        bad = jnp.max(jnp.abs(cnt - jnp.float32(K_NN))) > 0.0
        mref[...] = m

        @pl.when(bad)
        def _full_chain():
            mm = jnp.max(d, axis=1, keepdims=True)
            for _ in range(K_NN - 1):
                mm = jnp.max(jnp.where(d < mm, d, neg_inf),
                             axis=1, keepdims=True)
            mref[...] = mm

        mask = (d >= mref[...]).astype(jnp.float32)

        g = jax.lax.dot_general(x_all_prev_ref[...], mask,
                                (((1,), (1,)), ((), ())),
                                preferred_element_type=jnp.float32)
        g = g * jnp.float32(1.0 / K_NN)

        o = jax.lax.dot_general(w1_ref[...], g, (((1,), (0,)), ((), ())),
                                preferred_element_type=jnp.float32)
        o = o + jax.lax.dot_general(wd_ref[...], x_prev_ref[...],
                                    (((1,), (0,)), ((), ())),
                                    preferred_element_type=jnp.float32)
        out_ref[...] = o + bias_ref[...]


def kernel(x, W, b):
    B, C, N = x.shape
    O = W.shape[0]
    Wm = W[:, :, 0, 0]
    w1 = Wm[:, :C]
    wd = Wm[:, C:] - w1
    bias = b[:, None]

    R = min(512, N)
    nb = N // R
    nblocks = B * nb
    grid = (nblocks + 1,)

    def cur_blk(t):
        f = jnp.minimum(t, nblocks - 1)
        return f // nb, 0, f % nb

    def prev_blk(t):
        g = jnp.maximum(t - 1, 0)
        return g // nb, 0, g % nb

    return pl.pallas_call(
        _knn_feat_kernel,
        grid=grid,
        in_specs=[
            pl.BlockSpec((None, C, R), cur_blk),
            pl.BlockSpec((None, C, N), lambda t: (jnp.minimum(t, nblocks - 1) // nb, 0, 0)),
            pl.BlockSpec((None, C, R), prev_blk),
            pl.BlockSpec((None, C, N), lambda t: (jnp.maximum(t - 1, 0) // nb, 0, 0)),
            pl.BlockSpec((O, C), lambda t: (0, 0)),
            pl.BlockSpec((O, C), lambda t: (0, 0)),
            pl.BlockSpec((O, 1), lambda t: (0, 0)),
        ],
        out_specs=pl.BlockSpec((None, O, R), prev_blk),
        out_shape=jax.ShapeDtypeStruct((B, O, N), jnp.float32),
        scratch_shapes=[pltpu.VMEM((2, R, N), jnp.float32),
                        pltpu.VMEM((R, 1), jnp.float32)],
    )(x, x, x, x, w1, wd, bias)
